# baseline (speedup 1.0000x reference)
import functools

import jax
import jax.numpy as jnp
import numpy as np
from jax import lax
from jax.experimental import pallas as pl
from jax.experimental.pallas import tpu as pltpu
from jax.experimental.pallas import tpu_sc as plsc

KNN = 16
NB = 256
F32_MIN = float(jnp.finfo(jnp.float32).min)


def _knn_kernel(pc_ref, pcfull_ref, pcT_ref, idx_ref):
    b = pl.program_id(0)
    N = pcT_ref.shape[2]

    pc_all = pcT_ref[0]
    xyz_blk = pc_ref[0, :, 0:3]
    xyz_all = pcfull_ref[0, :, 0:3]

    inner = lax.dot_general(xyz_blk, xyz_all, (((1,), (1,)), ((), ())),
                            preferred_element_type=jnp.float32)

    sq_all = (pc_all[0:1, :] * pc_all[0:1, :] + pc_all[1:2, :] * pc_all[1:2, :]) \
        + pc_all[2:3, :] * pc_all[2:3, :]
    sq_blk = (xyz_blk[:, 0:1] * xyz_blk[:, 0:1] + xyz_blk[:, 1:2] * xyz_blk[:, 1:2]) \
        + xyz_blk[:, 2:3] * xyz_blk[:, 2:3]

    neg = -((sq_blk - 2.0 * inner) + sq_all)
    iota = lax.broadcasted_iota(jnp.int32, (NB, N), 1)

    for k in range(KNN):
        m = jnp.max(neg, axis=1)
        wins = neg == m[:, None]
        ik = jnp.min(jnp.where(wins, iota, N), axis=1)
        idx_ref[0, :, k] = ik + b * N
        neg = jnp.where(iota == ik[:, None], F32_MIN, neg)


def _run_knn(pc, pcT):
    B, N, _ = pc.shape
    grid = (B, N // NB)
    return pl.pallas_call(
        _knn_kernel,
        grid=grid,
        in_specs=[
            pl.BlockSpec((1, NB, 6), lambda b, n: (b, n, 0)),
            pl.BlockSpec((1, N, 6), lambda b, n: (b, 0, 0)),
            pl.BlockSpec((1, 6, N), lambda b, n: (b, 0, 0)),
        ],
        out_specs=pl.BlockSpec((1, NB, KNN), lambda b, n: (b, n, 0)),
        out_shape=jax.ShapeDtypeStruct((B, N, KNN), jnp.int32),
    )(pc, pc, pcT)


def _taylor_conv1_kernel(g1_ref, pc_ref, S_ref, P_ref, b1r_ref,
                         e_ref, w2_ref, b2_ref, tay_ref, f_ref):
    gflat = g1_ref[0]
    xyz = jax.lax.dot_general(gflat, S_ref[...], (((1,), (0,)), ((), ())),
                              precision=lax.Precision.HIGHEST,
                              preferred_element_type=jnp.float32)
    cx = pc_ref[0, :, 0:1]; cy = pc_ref[0, :, 1:2]; cz = pc_ref[0, :, 2:3]
    X = xyz[:, 0:KNN] - cx
    Y = xyz[:, KNN:2 * KNN] - cy
    Z = xyz[:, 2 * KNN:3 * KNN] - cz
    ones = jnp.ones_like(X)
    XY = X * Y; XZ = X * Z; YZ = Y * Z
    XX = X * X; YY = Y * Y; ZZ = Z * Z
    terms = jnp.concatenate(
        [ones, X, Y, Z, XY, XZ, YZ, XX, YY, ZZ,
         XY * Z, XX * Y, XX * Z, X * YY, YY * Z,
         X * ZZ, Y * ZZ, XX * X, YY * Y, ZZ * Z], axis=1)
    tayflat = jnp.dot(terms, P_ref[...],
                      preferred_element_type=jnp.float32) + b1r_ref[0]
    tay_ref[0] = tayflat
    f_ref[0] = _conv_body(g1_ref[0], tayflat, e_ref, w2_ref, b2_ref)


def _run_taylor_conv1(g1flat, pc, S, P, b1rep, E, w2r3, b2, Cout):
    B, N, _ = pc.shape
    grid = (B, N // NB)
    Cpad = g1flat.shape[2] // KNN
    return pl.pallas_call(
        _taylor_conv1_kernel,
        grid=grid,
        in_specs=[
            pl.BlockSpec((1, NB, KNN * 16), lambda b, n: (b, n, 0)),
            pl.BlockSpec((1, NB, 6), lambda b, n: (b, n, 0)),
            pl.BlockSpec((KNN * 16, 3 * KNN), lambda b, n: (0, 0)),
            pl.BlockSpec((20 * KNN, 12 * KNN), lambda b, n: (0, 0)),
            pl.BlockSpec((1, 12 * KNN), lambda b, n: (0, 0)),
            pl.BlockSpec((3, KNN * 12, KNN * Cpad), lambda b, n: (0, 0, 0)),
            pl.BlockSpec((3, KNN * Cpad, Cout), lambda b, n: (0, 0, 0)),
            pl.BlockSpec((1, Cout), lambda b, n: (0, 0)),
        ],
        out_specs=[
            pl.BlockSpec((1, NB, 12 * KNN), lambda b, n: (b, n, 0)),
            pl.BlockSpec((1, NB, Cout), lambda b, n: (b, n, 0)),
        ],
        out_shape=[
            jax.ShapeDtypeStruct((B, N, 12 * KNN), jnp.float32),
            jax.ShapeDtypeStruct((B, N, Cout), jnp.float32),
        ],
    )(g1flat, pc, S, P, b1rep, E, w2r3, b2)


def _coord_selector():
    S = np.zeros((KNN * 16, 3 * KNN), np.float32)
    for c in range(3):
        for k in range(KNN):
            S[k * 16 + c, c * KNN + k] = 1.0
    return jnp.asarray(S)


def _sc_gather(table, idx_flat):
    R = idx_flat.shape[0]
    C = table.shape[1]
    NW = 32
    per_w = R // NW
    CH = 128
    n_ch = per_w // CH
    idx2d = idx_flat.reshape(R // CH, CH)
    mesh = plsc.VectorSubcoreMesh(core_axis_name="c", subcore_axis_name="s")

    @functools.partial(
        pl.kernel, mesh=mesh,
        out_type=jax.ShapeDtypeStruct((R, C), jnp.float32),
        compiler_params=pltpu.CompilerParams(use_tc_tiling_on_sc=False),
        scratch_types=[
            pltpu.VMEM((n_ch, CH), jnp.int32),
            pltpu.VMEM((CH, C), jnp.float32),
            pltpu.VMEM((CH, C), jnp.float32),
            pltpu.SemaphoreType.DMA,
            pltpu.SemaphoreType.DMA,
        ],
    )
    def k(table_hbm, idx_hbm, out_hbm, idx_v, rows0, rows1, sem0, sem1):
        wid = lax.axis_index("s") * 2 + lax.axis_index("c")
        base = wid * per_w
        pltpu.sync_copy(idx_hbm.at[pl.ds(wid * n_ch, n_ch)], idx_v)
        pltpu.async_copy(table_hbm.at[idx_v.at[0]], rows0, sem0)

        def body(j, carry):
            c0 = 2 * j
            c1 = c0 + 1
            pltpu.async_copy(table_hbm.at[idx_v.at[c1]], rows1, sem1)
            pltpu.make_async_copy(table_hbm.at[idx_v.at[c0]], rows0, sem0).wait()
            pltpu.sync_copy(rows0, out_hbm.at[pl.ds(base + c0 * CH, CH)])

            @pl.when(c0 + 2 < n_ch)
            def _():
                pltpu.async_copy(table_hbm.at[idx_v.at[c0 + 2]], rows0, sem0)

            pltpu.make_async_copy(table_hbm.at[idx_v.at[c1]], rows1, sem1).wait()
            pltpu.sync_copy(rows1, out_hbm.at[pl.ds(base + c1 * CH, CH)])
            return carry

        lax.fori_loop(0, n_ch // 2, body, 0)

    return k(table, idx2d)


def _conv_body(Gflat, tayflat, e_ref, w2_ref, b2_ref):
    acc = None
    for t in range(3):
        texp = jnp.dot(tayflat, e_ref[t],
                       preferred_element_type=jnp.float32)
        y = jnp.dot(Gflat * texp, w2_ref[t],
                    preferred_element_type=jnp.float32)
        acc = y if acc is None else acc + y
    return jnp.maximum(acc + b2_ref[0], 0.0)


def _make_conv_kernel(Cpad):
    def conv_kernel(g_ref, tay_ref, e_ref, w2_ref, b2_ref, out_ref):
        out_ref[0] = _conv_body(g_ref[0], tay_ref[0], e_ref, w2_ref, b2_ref)
    return conv_kernel


def _run_conv(gflat, tay, E, w2r3, b2, B, N, Cout):
    grid = (B, N // NB)
    Cpad = gflat.shape[2] // KNN
    return pl.pallas_call(
        _make_conv_kernel(Cpad),
        grid=grid,
        in_specs=[
            pl.BlockSpec((1, NB, KNN * Cpad), lambda b, n: (b, n, 0)),
            pl.BlockSpec((1, NB, KNN * 12), lambda b, n: (b, n, 0)),
            pl.BlockSpec((3, KNN * 12, KNN * Cpad), lambda b, n: (0, 0, 0)),
            pl.BlockSpec((3, KNN * Cpad, Cout), lambda b, n: (0, 0, 0)),
            pl.BlockSpec((1, Cout), lambda b, n: (0, 0)),
        ],
        out_specs=pl.BlockSpec((1, NB, Cout), lambda b, n: (b, n, 0)),
        out_shape=jax.ShapeDtypeStruct((B, N, Cout), jnp.float32),
    )(gflat, tay, E, w2r3, b2)


def _top2_kernel(cat_ref, out_ref):
    v = cat_ref[0]
    Cc, N = v.shape
    iota = lax.broadcasted_iota(jnp.int32, (Cc, N), 1)
    m1 = jnp.max(v, axis=1)
    wins = v == m1[:, None]
    n1 = jnp.min(jnp.where(wins, iota, N), axis=1)
    m2 = jnp.max(jnp.where(iota == n1[:, None], F32_MIN, v), axis=1)
    out_ref[0, :, 0] = m1
    out_ref[0, :, 1] = m2


def _run_top2(cat):
    B, Cc, N = cat.shape
    return pl.pallas_call(
        _top2_kernel,
        grid=(B,),
        in_specs=[pl.BlockSpec((1, Cc, N), lambda b: (b, 0, 0))],
        out_specs=pl.BlockSpec((1, Cc, 2), lambda b: (b, 0, 0)),
        out_shape=jax.ShapeDtypeStruct((B, Cc, 2), jnp.float32),
    )(cat)


def _w2_rearrange(w2, C, Cpad):
    Cout = w2.shape[0]
    w4 = jnp.transpose(w2.astype(jnp.bfloat16).reshape(Cout, C, 3, KNN),
                       (2, 3, 1, 0))
    w4 = jnp.pad(w4, ((0, 0), (0, 0), (0, Cpad - C), (0, 0)))
    return w4.reshape(3, KNN * Cpad, Cout)


def _expand_mats(layer, Cpad):
    E = np.zeros((3, KNN * 12, KNN * Cpad), np.float32)
    for t in range(3):
        for k in range(KNN):
            E[t, k * 12 + 3 * layer + t, k * Cpad:(k + 1) * Cpad] = 1.0
    return jnp.asarray(E, dtype=jnp.bfloat16)


def _taylor_pattern(w1all):
    eye = jnp.eye(KNN, dtype=jnp.float32)
    P = w1all[:, None, None, :] * eye[None, :, :, None]
    return P.reshape(20 * KNN, 12 * KNN)


def kernel(pc, w1_1, b1_1, w2_1, b2_1, w1_2, b1_2, w2_2, b2_2,
           w1_3, b1_3, w2_3, b2_3, w1_4, b1_4, w2_4, b2_4):
    B, N, _ = pc.shape
    pcT = jnp.transpose(pc, (0, 2, 1))

    w1all = jnp.concatenate([w1_1.T, w1_2.T, w1_3.T, w1_4.T], axis=1)
    b1all = jnp.concatenate([b1_1, b1_2, b1_3, b1_4])[None, :]

    idx_g = _run_knn(pc, pcT)
    idx_flat = idx_g.reshape(B * N * KNN)

    pc_pad = jnp.pad(pc, ((0, 0), (0, 0), (0, 10)))
    g1rows = _sc_gather(pc_pad.reshape(B * N, 16), idx_flat)

    cins = (6, 32, 64, 128)
    couts = (32, 64, 128, 256)
    w2s = (w2_1, w2_2, w2_3, w2_4)
    b2s = (b2_1, b2_2, b2_3, b2_4)

    tay, f1 = _run_taylor_conv1(
        g1rows.reshape(B, N, KNN * 16), pc, _coord_selector(),
        _taylor_pattern(w1all), jnp.tile(b1all[0], KNN)[None, :],
        _expand_mats(0, 16), _w2_rearrange(w2_1, 6, 16),
        b2_1[None, :], couts[0])

    feats = [f1]
    grows = _sc_gather(f1.reshape(B * N, couts[0]), idx_flat)
    for l in range(1, 4):
        C, Cout = cins[l], couts[l]
        Cpad = grows.shape[1]
        w2r3 = _w2_rearrange(w2s[l], C, Cpad)
        E = _expand_mats(l, Cpad)
        gflat = grows.reshape(B, N, KNN * Cpad)
        f = _run_conv(gflat, tay, E, w2r3, b2s[l][None, :], B, N, Cout)
        feats.append(f)
        if l < 3:
            grows = _sc_gather(f.reshape(B * N, Cout), idx_flat)

    cat = jnp.concatenate([jnp.transpose(f, (0, 2, 1)) for f in feats], axis=1)
    top2 = _run_top2(cat)
    return top2.reshape(B, 2 * cat.shape[1]), cat


# speedup vs baseline: 1.0369x; 1.0369x over previous
import functools

import jax
import jax.numpy as jnp
import numpy as np
from jax import lax
from jax.experimental import pallas as pl
from jax.experimental.pallas import tpu as pltpu
from jax.experimental.pallas import tpu_sc as plsc

KNN = 16
NB = 256
F32_MIN = float(jnp.finfo(jnp.float32).min)


NBK = 512


def _knn_kernel(pc_ref, pcfull_ref, pcT_ref, idx_ref):
    b = pl.program_id(0)
    N = pcT_ref.shape[2]

    pc_all = pcT_ref[0]
    xyz_blk = pc_ref[0, :, 0:3]
    xyz_all = pcfull_ref[0, :, 0:3]

    inner = lax.dot_general(xyz_blk, xyz_all, (((1,), (1,)), ((), ())),
                            preferred_element_type=jnp.float32)

    sq_all = (pc_all[0:1, :] * pc_all[0:1, :] + pc_all[1:2, :] * pc_all[1:2, :]) \
        + pc_all[2:3, :] * pc_all[2:3, :]
    sq_blk = (xyz_blk[:, 0:1] * xyz_blk[:, 0:1] + xyz_blk[:, 1:2] * xyz_blk[:, 1:2]) \
        + xyz_blk[:, 2:3] * xyz_blk[:, 2:3]

    neg = -((sq_blk - 2.0 * inner) + sq_all)
    iota = lax.broadcasted_iota(jnp.int32, (NBK, N), 1)

    for k in range(KNN):
        m = jnp.max(neg, axis=1)
        wins = neg == m[:, None]
        ik = jnp.min(jnp.where(wins, iota, N), axis=1)
        idx_ref[0, :, k] = ik + b * N
        neg = jnp.where(iota == ik[:, None], F32_MIN, neg)


def _run_knn(pc, pcT):
    B, N, _ = pc.shape
    grid = (B, N // NBK)
    return pl.pallas_call(
        _knn_kernel,
        grid=grid,
        in_specs=[
            pl.BlockSpec((1, NBK, 6), lambda b, n: (b, n, 0)),
            pl.BlockSpec((1, N, 6), lambda b, n: (b, 0, 0)),
            pl.BlockSpec((1, 6, N), lambda b, n: (b, 0, 0)),
        ],
        out_specs=pl.BlockSpec((1, NBK, KNN), lambda b, n: (b, n, 0)),
        out_shape=jax.ShapeDtypeStruct((B, N, KNN), jnp.int32),
    )(pc, pc, pcT)


def _taylor_conv1_kernel(g1_ref, pc_ref, S_ref, P_ref, b1r_ref,
                         e_ref, w2_ref, b2_ref, tay_ref, f_ref):
    gflat = g1_ref[0]
    xyz = jax.lax.dot_general(gflat, S_ref[...], (((1,), (0,)), ((), ())),
                              precision=lax.Precision.HIGHEST,
                              preferred_element_type=jnp.float32)
    cx = pc_ref[0, :, 0:1]; cy = pc_ref[0, :, 1:2]; cz = pc_ref[0, :, 2:3]
    X = xyz[:, 0:KNN] - cx
    Y = xyz[:, KNN:2 * KNN] - cy
    Z = xyz[:, 2 * KNN:3 * KNN] - cz
    ones = jnp.ones_like(X)
    XY = X * Y; XZ = X * Z; YZ = Y * Z
    XX = X * X; YY = Y * Y; ZZ = Z * Z
    terms = jnp.concatenate(
        [ones, X, Y, Z, XY, XZ, YZ, XX, YY, ZZ,
         XY * Z, XX * Y, XX * Z, X * YY, YY * Z,
         X * ZZ, Y * ZZ, XX * X, YY * Y, ZZ * Z], axis=1)
    tayflat = jnp.dot(terms, P_ref[...],
                      preferred_element_type=jnp.float32) + b1r_ref[0]
    tay_ref[0] = tayflat
    f_ref[0] = _conv_body(g1_ref[0], tayflat, e_ref, w2_ref, b2_ref)


def _run_taylor_conv1(g1flat, pc, S, P, b1rep, E, w2r3, b2, Cout):
    B, N, _ = pc.shape
    grid = (B, N // NB)
    Cpad = g1flat.shape[2] // KNN
    return pl.pallas_call(
        _taylor_conv1_kernel,
        grid=grid,
        in_specs=[
            pl.BlockSpec((1, NB, KNN * 16), lambda b, n: (b, n, 0)),
            pl.BlockSpec((1, NB, 6), lambda b, n: (b, n, 0)),
            pl.BlockSpec((KNN * 16, 3 * KNN), lambda b, n: (0, 0)),
            pl.BlockSpec((20 * KNN, 12 * KNN), lambda b, n: (0, 0)),
            pl.BlockSpec((1, 12 * KNN), lambda b, n: (0, 0)),
            pl.BlockSpec((3, KNN * 12, KNN * Cpad), lambda b, n: (0, 0, 0)),
            pl.BlockSpec((3, KNN * Cpad, Cout), lambda b, n: (0, 0, 0)),
            pl.BlockSpec((1, Cout), lambda b, n: (0, 0)),
        ],
        out_specs=[
            pl.BlockSpec((1, NB, 12 * KNN), lambda b, n: (b, n, 0)),
            pl.BlockSpec((1, NB, Cout), lambda b, n: (b, n, 0)),
        ],
        out_shape=[
            jax.ShapeDtypeStruct((B, N, 12 * KNN), jnp.float32),
            jax.ShapeDtypeStruct((B, N, Cout), jnp.float32),
        ],
    )(g1flat, pc, S, P, b1rep, E, w2r3, b2)


def _coord_selector():
    S = np.zeros((KNN * 16, 3 * KNN), np.float32)
    for c in range(3):
        for k in range(KNN):
            S[k * 16 + c, c * KNN + k] = 1.0
    return jnp.asarray(S)


def _sc_gather(table, idx_flat):
    R = idx_flat.shape[0]
    C = table.shape[1]
    NW = 32
    per_w = R // NW
    CH = 128
    n_ch = per_w // CH
    idx2d = idx_flat.reshape(R // CH, CH)
    mesh = plsc.VectorSubcoreMesh(core_axis_name="c", subcore_axis_name="s")

    @functools.partial(
        pl.kernel, mesh=mesh,
        out_type=jax.ShapeDtypeStruct((R, C), jnp.float32),
        compiler_params=pltpu.CompilerParams(use_tc_tiling_on_sc=False),
        scratch_types=[
            pltpu.VMEM((n_ch, CH), jnp.int32),
            pltpu.VMEM((CH, C), jnp.float32),
            pltpu.VMEM((CH, C), jnp.float32),
            pltpu.SemaphoreType.DMA,
            pltpu.SemaphoreType.DMA,
        ],
    )
    def k(table_hbm, idx_hbm, out_hbm, idx_v, rows0, rows1, sem0, sem1):
        wid = lax.axis_index("s") * 2 + lax.axis_index("c")
        base = wid * per_w
        pltpu.sync_copy(idx_hbm.at[pl.ds(wid * n_ch, n_ch)], idx_v)
        pltpu.async_copy(table_hbm.at[idx_v.at[0]], rows0, sem0)

        def body(j, carry):
            c0 = 2 * j
            c1 = c0 + 1
            pltpu.async_copy(table_hbm.at[idx_v.at[c1]], rows1, sem1)
            pltpu.make_async_copy(table_hbm.at[idx_v.at[c0]], rows0, sem0).wait()
            pltpu.sync_copy(rows0, out_hbm.at[pl.ds(base + c0 * CH, CH)])

            @pl.when(c0 + 2 < n_ch)
            def _():
                pltpu.async_copy(table_hbm.at[idx_v.at[c0 + 2]], rows0, sem0)

            pltpu.make_async_copy(table_hbm.at[idx_v.at[c1]], rows1, sem1).wait()
            pltpu.sync_copy(rows1, out_hbm.at[pl.ds(base + c1 * CH, CH)])
            return carry

        lax.fori_loop(0, n_ch // 2, body, 0)

    return k(table, idx2d)


def _conv_body(Gflat, tayflat, e_ref, w2_ref, b2_ref):
    acc = None
    for t in range(3):
        texp = jnp.dot(tayflat, e_ref[t],
                       preferred_element_type=jnp.float32)
        y = jnp.dot(Gflat * texp, w2_ref[t],
                    preferred_element_type=jnp.float32)
        acc = y if acc is None else acc + y
    return jnp.maximum(acc + b2_ref[0], 0.0)


def _make_conv_kernel(Cpad):
    def conv_kernel(g_ref, tay_ref, e_ref, w2_ref, b2_ref, out_ref):
        out_ref[0] = _conv_body(g_ref[0], tay_ref[0], e_ref, w2_ref, b2_ref)
    return conv_kernel


def _run_conv(gflat, tay, E, w2r3, b2, B, N, Cout):
    grid = (B, N // NB)
    Cpad = gflat.shape[2] // KNN
    return pl.pallas_call(
        _make_conv_kernel(Cpad),
        grid=grid,
        in_specs=[
            pl.BlockSpec((1, NB, KNN * Cpad), lambda b, n: (b, n, 0)),
            pl.BlockSpec((1, NB, KNN * 12), lambda b, n: (b, n, 0)),
            pl.BlockSpec((3, KNN * 12, KNN * Cpad), lambda b, n: (0, 0, 0)),
            pl.BlockSpec((3, KNN * Cpad, Cout), lambda b, n: (0, 0, 0)),
            pl.BlockSpec((1, Cout), lambda b, n: (0, 0)),
        ],
        out_specs=pl.BlockSpec((1, NB, Cout), lambda b, n: (b, n, 0)),
        out_shape=jax.ShapeDtypeStruct((B, N, Cout), jnp.float32),
    )(gflat, tay, E, w2r3, b2)


def _top2_kernel(cat_ref, out_ref):
    v = cat_ref[0]
    Cc, N = v.shape
    iota = lax.broadcasted_iota(jnp.int32, (Cc, N), 1)
    m1 = jnp.max(v, axis=1)
    wins = v == m1[:, None]
    n1 = jnp.min(jnp.where(wins, iota, N), axis=1)
    m2 = jnp.max(jnp.where(iota == n1[:, None], F32_MIN, v), axis=1)
    out_ref[0, :, 0] = m1
    out_ref[0, :, 1] = m2


def _run_top2(cat):
    B, Cc, N = cat.shape
    return pl.pallas_call(
        _top2_kernel,
        grid=(B,),
        in_specs=[pl.BlockSpec((1, Cc, N), lambda b: (b, 0, 0))],
        out_specs=pl.BlockSpec((1, Cc, 2), lambda b: (b, 0, 0)),
        out_shape=jax.ShapeDtypeStruct((B, Cc, 2), jnp.float32),
    )(cat)


def _w2_rearrange(w2, C, Cpad):
    Cout = w2.shape[0]
    w4 = jnp.transpose(w2.astype(jnp.bfloat16).reshape(Cout, C, 3, KNN),
                       (2, 3, 1, 0))
    w4 = jnp.pad(w4, ((0, 0), (0, 0), (0, Cpad - C), (0, 0)))
    return w4.reshape(3, KNN * Cpad, Cout)


def _expand_mats(layer, Cpad):
    E = np.zeros((3, KNN * 12, KNN * Cpad), np.float32)
    for t in range(3):
        for k in range(KNN):
            E[t, k * 12 + 3 * layer + t, k * Cpad:(k + 1) * Cpad] = 1.0
    return jnp.asarray(E, dtype=jnp.bfloat16)


def _taylor_pattern(w1all):
    eye = jnp.eye(KNN, dtype=jnp.float32)
    P = w1all[:, None, None, :] * eye[None, :, :, None]
    return P.reshape(20 * KNN, 12 * KNN)


def kernel(pc, w1_1, b1_1, w2_1, b2_1, w1_2, b1_2, w2_2, b2_2,
           w1_3, b1_3, w2_3, b2_3, w1_4, b1_4, w2_4, b2_4):
    B, N, _ = pc.shape
    pcT = jnp.transpose(pc, (0, 2, 1))

    w1all = jnp.concatenate([w1_1.T, w1_2.T, w1_3.T, w1_4.T], axis=1)
    b1all = jnp.concatenate([b1_1, b1_2, b1_3, b1_4])[None, :]

    idx_g = _run_knn(pc, pcT)
    idx_flat = idx_g.reshape(B * N * KNN)

    pc_pad = jnp.pad(pc, ((0, 0), (0, 0), (0, 10)))
    g1rows = _sc_gather(pc_pad.reshape(B * N, 16), idx_flat)

    cins = (6, 32, 64, 128)
    couts = (32, 64, 128, 256)
    w2s = (w2_1, w2_2, w2_3, w2_4)
    b2s = (b2_1, b2_2, b2_3, b2_4)

    tay, f1 = _run_taylor_conv1(
        g1rows.reshape(B, N, KNN * 16), pc, _coord_selector(),
        _taylor_pattern(w1all), jnp.tile(b1all[0], KNN)[None, :],
        _expand_mats(0, 16), _w2_rearrange(w2_1, 6, 16),
        b2_1[None, :], couts[0])

    feats = [f1]
    grows = _sc_gather(f1.reshape(B * N, couts[0]), idx_flat)
    for l in range(1, 4):
        C, Cout = cins[l], couts[l]
        Cpad = grows.shape[1]
        w2r3 = _w2_rearrange(w2s[l], C, Cpad)
        E = _expand_mats(l, Cpad)
        gflat = grows.reshape(B, N, KNN * Cpad)
        f = _run_conv(gflat, tay, E, w2r3, b2s[l][None, :], B, N, Cout)
        feats.append(f)
        if l < 3:
            grows = _sc_gather(f.reshape(B * N, Cout), idx_flat)

    cat = jnp.concatenate([jnp.transpose(f, (0, 2, 1)) for f in feats], axis=1)
    top2 = _run_top2(cat)
    return top2.reshape(B, 2 * cat.shape[1]), cat
